# Initial kernel scaffold; baseline (speedup 1.0000x reference)
import jax
import jax.numpy as jnp
from jax.experimental import pallas as pl


def kernel(user_emb, social_weight, edge_index, adj_values):
    raise NotImplementedError("write your pallas kernel here")


# speedup vs baseline: 4.3443x; 4.3443x over previous
import functools

import jax
import jax.numpy as jnp
from jax import lax
from jax.experimental import pallas as pl
from jax.experimental.pallas import tpu as pltpu
from jax.experimental.pallas import tpu_sc as plsc

N_USERS = 100000
D = 64
N_EDGES = 1600000

NSC = 2
NTILE = 16
HALF = N_USERS // NSC
NPASS = 2
P = HALF // NPASS
EPT = N_EDGES // NTILE
C = 2000
NVR = C // 16
GB = 128


def _mm_body(x_ref, w_ref, o_ref):
    o_ref[...] = jnp.dot(x_ref[...], w_ref[...],
                         preferred_element_type=jnp.float32)


def _matmul(x, w):
    BM = 2000
    return pl.pallas_call(
        _mm_body,
        grid=(N_USERS // BM,),
        in_specs=[pl.BlockSpec((BM, D), lambda i: (i, 0)),
                  pl.BlockSpec((D, D), lambda i: (0, 0))],
        out_specs=pl.BlockSpec((BM, D), lambda i: (i, 0)),
        out_shape=jax.ShapeDtypeStruct((N_USERS, D), jnp.float32),
    )(x, w)


_mesh = plsc.VectorSubcoreMesh(core_axis_name="c", subcore_axis_name="s")


@functools.partial(
    pl.kernel,
    mesh=_mesh,
    compiler_params=pltpu.CompilerParams(needs_layout_passes=False,
                                         use_tc_tiling_on_sc=False),
    out_type=jax.ShapeDtypeStruct((N_USERS, D), jnp.float32),
    scratch_types=[
        pltpu.VMEM((C,), jnp.int32),
        pltpu.VMEM((C,), jnp.int32),
        pltpu.VMEM((C,), jnp.float32),
        pltpu.VMEM((C + GB,), jnp.int32),
        pltpu.VMEM((C + GB,), jnp.int32),
        pltpu.VMEM((C + GB,), jnp.float32),
        pltpu.VMEM((GB,), jnp.int32),
        pltpu.VMEM((GB,), jnp.int32),
        pltpu.VMEM((GB, D), jnp.float32),
        pltpu.VMEM_SHARED((P, D), jnp.float32),
        pltpu.SemaphoreType.DMA,
    ],
)
def _sc_spmm(weighted, user_emb, rows, cols, vals, out,
             rows_v, cols_v, vals_v, ccol, crow, cval,
             gidx, sidx, gbuf, acc, sem):
    cid = lax.axis_index("c")
    sid = lax.axis_index("s")

    for p in range(NPASS):
        lo = cid * HALF + p * P

        @pl.when(sid == 0)
        def _():
            pltpu.sync_copy(user_emb.at[pl.ds(lo, P)], acc)

        plsc.subcore_barrier()

        def chunk_body(ci, _, lo=lo):
            base = sid * EPT + ci * C
            pltpu.sync_copy(rows.at[pl.ds(base, C)], rows_v)
            pltpu.sync_copy(cols.at[pl.ds(base, C)], cols_v)
            pltpu.sync_copy(vals.at[pl.ds(base, C)], vals_v)

            def scan_body(k, cnt, lo=lo):
                rv = rows_v[pl.ds(k * 16, 16)]
                m = (rv >= lo) & (rv < lo + P)
                mi = m.astype(jnp.int32)
                pos = cnt + plsc.cumsum(mi) - 1
                plsc.store_scatter(ccol, [pos],
                                   cols_v[pl.ds(k * 16, 16)], mask=m)
                plsc.store_scatter(crow, [pos], rv - lo, mask=m)
                plsc.store_scatter(cval, [pos],
                                   vals_v[pl.ds(k * 16, 16)], mask=m)
                return cnt + plsc.all_reduce_population_count(m)[0]

            cnt = lax.fori_loop(0, NVR, scan_body, jnp.int32(0))

            pad = lax.iota(jnp.int32, 16)
            zero = jnp.zeros((16,), jnp.float32)
            for j in range(GB // 16):
                ccol[pl.ds(cnt + j * 16, 16)] = pad
                crow[pl.ds(cnt + j * 16, 16)] = pad
                cval[pl.ds(cnt + j * 16, 16)] = zero

            nb = (cnt + GB - 1) // GB

            def batch_body(b, _):
                for j in range(GB // 16):
                    gidx[pl.ds(j * 16, 16)] = ccol[pl.ds(b * GB + j * 16, 16)]
                    sidx[pl.ds(j * 16, 16)] = crow[pl.ds(b * GB + j * 16, 16)]
                pltpu.async_copy(weighted.at[gidx], gbuf, sem).wait()

                def rbody(r8, _):
                    vv = cval[pl.ds(b * GB + r8 * 16, 16)]
                    for u in range(16):
                        r = r8 * 16 + u
                        v = vv[u]
                        for q in range(D // 16):
                            gbuf[r, pl.ds(q * 16, 16)] = (
                                gbuf[r, pl.ds(q * 16, 16)] * v)
                    return jnp.int32(0)

                lax.fori_loop(0, GB // 16, rbody, jnp.int32(0))
                pltpu.sync_copy(gbuf, acc.at[sidx], add=True)
                return jnp.int32(0)

            lax.fori_loop(0, nb, batch_body, jnp.int32(0))
            return jnp.int32(0)

        lax.fori_loop(0, EPT // C, chunk_body, jnp.int32(0))

        plsc.subcore_barrier()

        @pl.when(sid == 0)
        def _():
            pltpu.sync_copy(acc, out.at[pl.ds(lo, P)])

        plsc.subcore_barrier()


def kernel(user_emb, social_weight, edge_index, adj_values):
    weighted = _matmul(user_emb, social_weight)
    rows = edge_index[0]
    cols = edge_index[1]
    return _sc_spmm(weighted, user_emb, rows, cols, adj_values)


# speedup vs baseline: 4.5533x; 1.0481x over previous
import functools

import jax
import jax.numpy as jnp
from jax import lax
from jax.experimental import pallas as pl
from jax.experimental.pallas import tpu as pltpu
from jax.experimental.pallas import tpu_sc as plsc

N_USERS = 100000
D = 64
N_EDGES = 1600000

NSC = 2
NTILE = 16
HALF = N_USERS // NSC
NPASS = 4
P = HALF // NPASS
EPT = N_EDGES // NTILE
C = 2000
NVR = C // 16
NCH = EPT // C
GB = 128


def _mm_body(x_ref, w_ref, o_ref):
    o_ref[...] = jnp.dot(x_ref[...], w_ref[...],
                         preferred_element_type=jnp.float32)


def _matmul(x, w):
    BM = 2000
    return pl.pallas_call(
        _mm_body,
        grid=(N_USERS // BM,),
        in_specs=[pl.BlockSpec((BM, D), lambda i: (i, 0)),
                  pl.BlockSpec((D, D), lambda i: (0, 0))],
        out_specs=pl.BlockSpec((BM, D), lambda i: (i, 0)),
        out_shape=jax.ShapeDtypeStruct((N_USERS, D), jnp.float32),
    )(x, w)


_mesh = plsc.VectorSubcoreMesh(core_axis_name="c", subcore_axis_name="s")


@functools.partial(
    pl.kernel,
    mesh=_mesh,
    compiler_params=pltpu.CompilerParams(needs_layout_passes=False,
                                         use_tc_tiling_on_sc=False),
    out_type=jax.ShapeDtypeStruct((N_USERS, D), jnp.float32),
    scratch_types=[
        [pltpu.VMEM((C,), jnp.int32)] * 2,
        [pltpu.VMEM((C,), jnp.int32)] * 2,
        [pltpu.VMEM((C,), jnp.float32)] * 2,
        pltpu.VMEM((C + GB,), jnp.int32),
        pltpu.VMEM((C + GB,), jnp.int32),
        pltpu.VMEM((C + GB,), jnp.float32),
        [pltpu.VMEM((GB,), jnp.int32)] * 2,
        [pltpu.VMEM((GB,), jnp.int32)] * 2,
        [pltpu.VMEM((GB, D), jnp.float32)] * 2,
        pltpu.VMEM_SHARED((P, D), jnp.float32),
        [pltpu.SemaphoreType.DMA] * 2,
        [pltpu.SemaphoreType.DMA] * 2,
        [pltpu.SemaphoreType.DMA] * 2,
    ],
)
def _sc_spmm(weighted, user_emb, rows, cols, vals, out,
             rows_v, cols_v, vals_v, ccol, crow, cval,
             gidx, sidx, gbuf, acc, sem_e, sem_g, sem_s):
    cid = lax.axis_index("c")
    sid = lax.axis_index("s")

    def fire_edges(ci, k):
        base = sid * EPT + ci * C
        pltpu.async_copy(rows.at[pl.ds(base, C)], rows_v[k], sem_e[k])
        pltpu.async_copy(cols.at[pl.ds(base, C)], cols_v[k], sem_e[k])
        pltpu.async_copy(vals.at[pl.ds(base, C)], vals_v[k], sem_e[k])

    def wait_edges(ci, k):
        base = sid * EPT + ci * C
        pltpu.make_async_copy(rows.at[pl.ds(base, C)], rows_v[k],
                              sem_e[k]).wait()
        pltpu.make_async_copy(cols.at[pl.ds(base, C)], cols_v[k],
                              sem_e[k]).wait()
        pltpu.make_async_copy(vals.at[pl.ds(base, C)], vals_v[k],
                              sem_e[k]).wait()

    def stage_and_fire_gather(b, k):
        for j in range(GB // 16):
            gidx[k][pl.ds(j * 16, 16)] = ccol[pl.ds(b * GB + j * 16, 16)]
            sidx[k][pl.ds(j * 16, 16)] = crow[pl.ds(b * GB + j * 16, 16)]
        pltpu.async_copy(weighted.at[gidx[k]], gbuf[k], sem_g[k])

    for p in range(NPASS):
        lo = cid * HALF + p * P

        @pl.when(sid == 0)
        def _():
            pltpu.sync_copy(user_emb.at[pl.ds(lo, P)], acc)

        plsc.subcore_barrier()

        fire_edges(0, 0)

        def chunk_pair(i, _, lo=lo):
            for k in range(2):
                ci = i * 2 + k

                @pl.when(ci + 1 < NCH)
                def _(ci=ci, k=k):
                    fire_edges(ci + 1, 1 - k)

                wait_edges(ci, k)

                def scan_body(j, cnt, lo=lo, k=k):
                    rv = rows_v[k][pl.ds(j * 16, 16)]
                    m = (rv >= lo) & (rv < lo + P)
                    mi = m.astype(jnp.int32)
                    pos = cnt + plsc.cumsum(mi) - 1
                    plsc.store_scatter(ccol, [pos],
                                       cols_v[k][pl.ds(j * 16, 16)], mask=m)
                    plsc.store_scatter(crow, [pos], rv - lo, mask=m)
                    plsc.store_scatter(cval, [pos],
                                       vals_v[k][pl.ds(j * 16, 16)], mask=m)
                    return cnt + plsc.all_reduce_population_count(m)[0]

                cnt = lax.fori_loop(0, NVR, scan_body, jnp.int32(0))

                pad = lax.iota(jnp.int32, 16)
                zero = jnp.zeros((16,), jnp.float32)
                for j in range(GB // 16):
                    ccol[pl.ds(cnt + j * 16, 16)] = pad
                    crow[pl.ds(cnt + j * 16, 16)] = pad
                    cval[pl.ds(cnt + j * 16, 16)] = zero

                nb = (cnt + GB - 1) // GB

                @pl.when(nb > 0)
                def _(nb=nb):
                    stage_and_fire_gather(0, 0)

                    def batch_pair(ii, _, nb=nb):
                        for kk in range(2):
                            b = ii * 2 + kk

                            @pl.when(b < nb)
                            def _(b=b, kk=kk):
                                @pl.when((b >= 1) & (b + 1 < nb))
                                def _(b=b, kk=kk):
                                    pltpu.make_async_copy(
                                        gbuf[1 - kk],
                                        acc.at[sidx[1 - kk]],
                                        sem_s[1 - kk]).wait()

                                @pl.when(b + 1 < nb)
                                def _(b=b, kk=kk):
                                    stage_and_fire_gather(b + 1, 1 - kk)

                                pltpu.make_async_copy(
                                    weighted.at[gidx[kk]], gbuf[kk],
                                    sem_g[kk]).wait()

                                def rbody(r8, _, b=b, kk=kk):
                                    vv = cval[pl.ds(b * GB + r8 * 16, 16)]
                                    for u in range(16):
                                        r = r8 * 16 + u
                                        v = vv[u]
                                        for q in range(D // 16):
                                            gbuf[kk][r, pl.ds(q * 16, 16)] = (
                                                gbuf[kk][r, pl.ds(q * 16, 16)]
                                                * v)
                                    return jnp.int32(0)

                                lax.fori_loop(0, GB // 16, rbody,
                                              jnp.int32(0))
                                pltpu.async_copy(gbuf[kk],
                                                 acc.at[sidx[kk]],
                                                 sem_s[kk], add=True)
                        return jnp.int32(0)

                    lax.fori_loop(0, (nb + 1) // 2, batch_pair, jnp.int32(0))

                    @pl.when(nb >= 1)
                    def _():
                        pltpu.make_async_copy(gbuf[0], acc.at[sidx[0]],
                                              sem_s[0]).wait()

                    @pl.when(nb >= 2)
                    def _():
                        pltpu.make_async_copy(gbuf[1], acc.at[sidx[1]],
                                              sem_s[1]).wait()
            return jnp.int32(0)

        lax.fori_loop(0, NCH // 2, chunk_pair, jnp.int32(0))

        plsc.subcore_barrier()

        @pl.when(sid == 0)
        def _():
            pltpu.sync_copy(acc, out.at[pl.ds(lo, P)])

        plsc.subcore_barrier()


def kernel(user_emb, social_weight, edge_index, adj_values):
    weighted = _matmul(user_emb, social_weight)
    rows = edge_index[0]
    cols = edge_index[1]
    return _sc_spmm(weighted, user_emb, rows, cols, adj_values)


# speedup vs baseline: 6.5385x; 1.4360x over previous
import functools

import jax
import jax.numpy as jnp
from jax import lax
from jax.experimental import pallas as pl
from jax.experimental.pallas import tpu as pltpu
from jax.experimental.pallas import tpu_sc as plsc

N_USERS = 100000
D = 64
N_EDGES = 1600000

NSC = 2
NTILE = 16
HALF = N_USERS // NSC
NPASS = 4
P = HALF // NPASS
EPT = N_EDGES // NTILE
C = 2000
NVR = C // 16
NCH = EPT // C
GB = 128


def _mm_body(x_ref, w_ref, o_ref):
    o_ref[...] = jnp.dot(x_ref[...], w_ref[...],
                         preferred_element_type=jnp.float32)


def _matmul(x, w):
    BM = 2000
    return pl.pallas_call(
        _mm_body,
        grid=(N_USERS // BM,),
        in_specs=[pl.BlockSpec((BM, D), lambda i: (i, 0)),
                  pl.BlockSpec((D, D), lambda i: (0, 0))],
        out_specs=pl.BlockSpec((BM, D), lambda i: (i, 0)),
        out_shape=jax.ShapeDtypeStruct((N_USERS, D), jnp.float32),
    )(x, w)


_mesh = plsc.VectorSubcoreMesh(core_axis_name="c", subcore_axis_name="s")


@functools.partial(
    pl.kernel,
    mesh=_mesh,
    compiler_params=pltpu.CompilerParams(needs_layout_passes=False,
                                         use_tc_tiling_on_sc=False),
    out_type=jax.ShapeDtypeStruct((N_USERS, D), jnp.float32),
    scratch_types=[
        [pltpu.VMEM((C,), jnp.int32)] * 2,
        [pltpu.VMEM((C,), jnp.int32)] * 2,
        [pltpu.VMEM((C,), jnp.float32)] * 2,
        pltpu.VMEM((C + GB,), jnp.int32),
        pltpu.VMEM((C + GB,), jnp.int32),
        pltpu.VMEM((C + GB,), jnp.float32),
        [pltpu.VMEM((GB,), jnp.int32)] * 2,
        [pltpu.VMEM((GB,), jnp.int32)] * 2,
        [pltpu.VMEM((GB, D), jnp.float32)] * 2,
        pltpu.VMEM_SHARED((P, D), jnp.float32),
        [pltpu.SemaphoreType.DMA] * 2,
        [pltpu.SemaphoreType.DMA] * 2,
        [pltpu.SemaphoreType.DMA] * 2,
    ],
)
def _sc_spmm(weighted, user_emb, rows, cols, vals, out,
             rows_v, cols_v, vals_v, ccol, crow, cval,
             gidx, sidx, gbuf, acc, sem_e, sem_g, sem_s):
    cid = lax.axis_index("c")
    sid = lax.axis_index("s")

    def fire_edges(ci, k):
        base = sid * EPT + ci * C
        pltpu.async_copy(rows.at[pl.ds(base, C)], rows_v[k], sem_e[k])
        pltpu.async_copy(cols.at[pl.ds(base, C)], cols_v[k], sem_e[k])
        pltpu.async_copy(vals.at[pl.ds(base, C)], vals_v[k], sem_e[k])

    def wait_edges(ci, k):
        base = sid * EPT + ci * C
        pltpu.make_async_copy(rows.at[pl.ds(base, C)], rows_v[k],
                              sem_e[k]).wait()
        pltpu.make_async_copy(cols.at[pl.ds(base, C)], cols_v[k],
                              sem_e[k]).wait()
        pltpu.make_async_copy(vals.at[pl.ds(base, C)], vals_v[k],
                              sem_e[k]).wait()

    def stage_and_fire_gather(b, k):
        for j in range(GB // 16):
            gidx[k][pl.ds(j * 16, 16)] = ccol[pl.ds(b * GB + j * 16, 16)]
            sidx[k][pl.ds(j * 16, 16)] = crow[pl.ds(b * GB + j * 16, 16)]
        pltpu.async_copy(weighted.at[gidx[k]], gbuf[k], sem_g[k])

    for p in range(NPASS):
        lo = cid * HALF + p * P

        @pl.when(sid == 0)
        def _():
            pltpu.sync_copy(user_emb.at[pl.ds(lo, P)], acc)

        plsc.subcore_barrier()

        fire_edges(0, 0)

        def chunk_pair(i, _, lo=lo):
            for k in range(2):
                ci = i * 2 + k

                @pl.when(ci + 1 < NCH)
                def _(ci=ci, k=k):
                    fire_edges(ci + 1, 1 - k)

                wait_edges(ci, k)

                def scan_body(j, cnt, lo=lo, k=k):
                    rv = rows_v[k][pl.ds(j * 16, 16)]
                    m = (rv >= lo) & (rv < lo + P)
                    mi = m.astype(jnp.int32)
                    pos = cnt + plsc.cumsum(mi) - 1
                    plsc.store_scatter(ccol, [pos],
                                       cols_v[k][pl.ds(j * 16, 16)], mask=m)
                    plsc.store_scatter(crow, [pos], rv - lo, mask=m)
                    plsc.store_scatter(cval, [pos],
                                       vals_v[k][pl.ds(j * 16, 16)], mask=m)
                    return pos[15] + 1

                cnt = lax.fori_loop(0, NVR, scan_body, jnp.int32(0),
                                    unroll=4)

                pad = lax.iota(jnp.int32, 16)
                zero = jnp.zeros((16,), jnp.float32)
                for j in range(GB // 16):
                    ccol[pl.ds(cnt + j * 16, 16)] = pad
                    crow[pl.ds(cnt + j * 16, 16)] = pad
                    cval[pl.ds(cnt + j * 16, 16)] = zero

                nb = (cnt + GB - 1) // GB

                @pl.when(nb > 0)
                def _(nb=nb):
                    stage_and_fire_gather(0, 0)

                    def batch_pair(ii, _, nb=nb):
                        for kk in range(2):
                            b = ii * 2 + kk

                            @pl.when(b < nb)
                            def _(b=b, kk=kk):
                                @pl.when((b >= 1) & (b + 1 < nb))
                                def _(b=b, kk=kk):
                                    pltpu.make_async_copy(
                                        gbuf[1 - kk],
                                        acc.at[sidx[1 - kk]],
                                        sem_s[1 - kk]).wait()

                                @pl.when(b + 1 < nb)
                                def _(b=b, kk=kk):
                                    stage_and_fire_gather(b + 1, 1 - kk)

                                pltpu.make_async_copy(
                                    weighted.at[gidx[kk]], gbuf[kk],
                                    sem_g[kk]).wait()

                                def rbody(r8, _, b=b, kk=kk):
                                    vv = cval[pl.ds(b * GB + r8 * 16, 16)]
                                    for u in range(16):
                                        r = r8 * 16 + u
                                        v = vv[u]
                                        for q in range(D // 16):
                                            gbuf[kk][r, pl.ds(q * 16, 16)] = (
                                                gbuf[kk][r, pl.ds(q * 16, 16)]
                                                * v)
                                    return jnp.int32(0)

                                lax.fori_loop(0, GB // 16, rbody,
                                              jnp.int32(0), unroll=2)
                                pltpu.async_copy(gbuf[kk],
                                                 acc.at[sidx[kk]],
                                                 sem_s[kk], add=True)
                        return jnp.int32(0)

                    lax.fori_loop(0, (nb + 1) // 2, batch_pair, jnp.int32(0))

                    @pl.when(nb >= 1)
                    def _():
                        pltpu.make_async_copy(gbuf[0], acc.at[sidx[0]],
                                              sem_s[0]).wait()

                    @pl.when(nb >= 2)
                    def _():
                        pltpu.make_async_copy(gbuf[1], acc.at[sidx[1]],
                                              sem_s[1]).wait()
            return jnp.int32(0)

        lax.fori_loop(0, NCH // 2, chunk_pair, jnp.int32(0))

        plsc.subcore_barrier()

        @pl.when(sid == 0)
        def _():
            pltpu.sync_copy(acc, out.at[pl.ds(lo, P)])

        plsc.subcore_barrier()


def kernel(user_emb, social_weight, edge_index, adj_values):
    weighted = _matmul(user_emb, social_weight)
    rows = edge_index[0]
    cols = edge_index[1]
    return _sc_spmm(weighted, user_emb, rows, cols, adj_values)
